# Initial kernel scaffold; baseline (speedup 1.0000x reference)
import jax
import jax.numpy as jnp
from jax.experimental import pallas as pl


def kernel(x, adj_norm, adj_spatial_norm, weight, ln_gamma, ln_beta):
    raise NotImplementedError("write your pallas kernel here")


# speedup vs baseline: 1.9916x; 1.9916x over previous
import jax
import jax.numpy as jnp
from jax.experimental import pallas as pl
from jax.experimental.pallas import tpu as pltpu

_G = 1000
_B = 512
_K = 64
_P = _B // 2


def _matmul_body(a_ref, x_ref, o_ref):
    o_ref[...] = jnp.dot(a_ref[...], x_ref[...],
                         preferred_element_type=jnp.float32)


def _fused_body(see_ref, seo_ref, soe_ref, soo_ref, w1_ref, w2_ref, w3_ref,
                gam_ref, bet_ref, x_ref, ng_ref, o_ref):
    gb = x_ref.shape[0]
    bk = x_ref.shape[1]
    rows = gb * _P

    x2 = x_ref[...].reshape(rows, 2 * _K)
    ng2 = ng_ref[...].reshape(rows, 2 * _K)

    lin = jnp.dot(x2, w1_ref[...], preferred_element_type=jnp.float32)
    lin = lin + jnp.dot(ng2, w2_ref[...], preferred_element_type=jnp.float32)
    p = jnp.dot(x2, w3_ref[...], preferred_element_type=jnp.float32)

    p3 = p.reshape(gb, _P, 2 * _K)
    pl_cat = jnp.concatenate([p3[i, :, :_K] for i in range(gb)], axis=1)
    ph_cat = jnp.concatenate([p3[i, :, _K:] for i in range(gb)], axis=1)

    nse = jnp.dot(see_ref[...], pl_cat, preferred_element_type=jnp.float32)
    nse = nse + jnp.dot(seo_ref[...], ph_cat,
                        preferred_element_type=jnp.float32)
    nso = jnp.dot(soe_ref[...], pl_cat, preferred_element_type=jnp.float32)
    nso = nso + jnp.dot(soo_ref[...], ph_cat,
                        preferred_element_type=jnp.float32)

    ns = jnp.stack([
        jnp.concatenate([nse[:, i * _K:(i + 1) * _K],
                         nso[:, i * _K:(i + 1) * _K]], axis=1)
        for i in range(gb)
    ])

    h = jax.nn.relu(lin + ns.reshape(rows, 2 * _K))
    r = x2 + h

    gam = gam_ref[...]
    bet = bet_ref[...]
    outs = []
    for half in (slice(0, _K), slice(_K, 2 * _K)):
        rh = r[:, half]
        mu = jnp.mean(rh, axis=1, keepdims=True)
        var = jnp.mean((rh - mu) ** 2, axis=1, keepdims=True)
        outs.append((rh - mu) * jax.lax.rsqrt(var + 1e-5) * gam + bet)
    o = jnp.concatenate(outs, axis=1)
    o_ref[...] = o.reshape(gb, bk)


def kernel(x, adj_norm, adj_spatial_norm, weight, ln_gamma, ln_beta):
    g, bk = x.shape
    cn = 2048
    ng = pl.pallas_call(
        _matmul_body,
        grid=(bk // cn,),
        in_specs=[
            pl.BlockSpec((g, g), lambda j: (0, 0)),
            pl.BlockSpec((g, cn), lambda j: (0, j)),
        ],
        out_specs=pl.BlockSpec((g, cn), lambda j: (0, j)),
        out_shape=jax.ShapeDtypeStruct((g, bk), jnp.float32),
        compiler_params=pltpu.CompilerParams(
            dimension_semantics=("arbitrary",),
        ),
    )(adj_norm, x)

    s = adj_spatial_norm
    see = s[0::2, 0::2]
    seo = s[0::2, 1::2]
    soe = s[1::2, 0::2]
    soo = s[1::2, 1::2]
    z = jnp.zeros((_K, _K), jnp.float32)
    w1, w2, w3 = weight[:_K], weight[_K:2 * _K], weight[2 * _K:]
    bd1 = jnp.block([[w1, z], [z, w1]])
    bd2 = jnp.block([[w2, z], [z, w2]])
    bd3 = jnp.block([[w3, z], [z, w3]])

    gblk = 8
    out = pl.pallas_call(
        _fused_body,
        grid=(g // gblk,),
        in_specs=[
            pl.BlockSpec((_P, _P), lambda i: (0, 0)),
            pl.BlockSpec((_P, _P), lambda i: (0, 0)),
            pl.BlockSpec((_P, _P), lambda i: (0, 0)),
            pl.BlockSpec((_P, _P), lambda i: (0, 0)),
            pl.BlockSpec((2 * _K, 2 * _K), lambda i: (0, 0)),
            pl.BlockSpec((2 * _K, 2 * _K), lambda i: (0, 0)),
            pl.BlockSpec((2 * _K, 2 * _K), lambda i: (0, 0)),
            pl.BlockSpec((1, _K), lambda i: (0, 0)),
            pl.BlockSpec((1, _K), lambda i: (0, 0)),
            pl.BlockSpec((gblk, bk), lambda i: (i, 0)),
            pl.BlockSpec((gblk, bk), lambda i: (i, 0)),
        ],
        out_specs=pl.BlockSpec((gblk, bk), lambda i: (i, 0)),
        out_shape=jax.ShapeDtypeStruct((g, bk), jnp.float32),
        compiler_params=pltpu.CompilerParams(
            dimension_semantics=("arbitrary",),
        ),
    )(see, seo, soe, soo, bd1, bd2, bd3,
      ln_gamma.reshape(1, _K), ln_beta.reshape(1, _K), x, ng)
    return out


# speedup vs baseline: 2.8788x; 1.4455x over previous
import jax
import jax.numpy as jnp
from jax.experimental import pallas as pl
from jax.experimental.pallas import tpu as pltpu

_G = 1000
_B = 512
_K = 64
_P = _B // 2
_L = 2 * _K


def _matmul_body(a_ref, x_ref, o_ref):
    o_ref[...] = jnp.dot(a_ref[...], x_ref[...],
                         preferred_element_type=jnp.float32)


def _fused_body(sbig_ref, bd1_ref, bd2_ref, bd3_ref, mred_ref,
                gam_ref, bet_ref, x_ref, ng_ref, o_ref):
    gb = x_ref.shape[0]
    bk = x_ref.shape[1]
    rows = gb * _P

    x2 = x_ref[...].reshape(rows, _L)
    ng2 = ng_ref[...].reshape(rows, _L)

    lin = jnp.dot(x2, bd1_ref[...], preferred_element_type=jnp.float32)
    lin = lin + jnp.dot(ng2, bd2_ref[...], preferred_element_type=jnp.float32)
    p = jnp.dot(x2, bd3_ref[...], preferred_element_type=jnp.float32)

    p3 = p.reshape(gb, _P, _L)
    sbig = sbig_ref[...]
    ns_list = []
    for i in range(gb):
        q = jnp.dot(sbig, p3[i], preferred_element_type=jnp.float32)
        lo = q[0:_P, 0:_K] + q[_P:2 * _P, _K:_L]
        hi = q[2 * _P:3 * _P, 0:_K] + q[3 * _P:4 * _P, _K:_L]
        ns_list.append(jnp.concatenate([lo, hi], axis=1))
    ns = jnp.stack(ns_list).reshape(rows, _L)

    h = jax.nn.relu(lin + ns)
    r = x2 + h

    mred = mred_ref[...]
    mu = jnp.dot(r, mred, preferred_element_type=jnp.float32)
    var = jnp.dot(r * r, mred, preferred_element_type=jnp.float32) - mu * mu
    o = (r - mu) * jax.lax.rsqrt(var + 1e-5) * gam_ref[...] + bet_ref[...]
    o_ref[...] = o.reshape(gb, bk)


def kernel(x, adj_norm, adj_spatial_norm, weight, ln_gamma, ln_beta):
    g, bk = x.shape
    cn = 2048
    ng = pl.pallas_call(
        _matmul_body,
        grid=(bk // cn,),
        in_specs=[
            pl.BlockSpec((g, g), lambda j: (0, 0)),
            pl.BlockSpec((g, cn), lambda j: (0, j)),
        ],
        out_specs=pl.BlockSpec((g, cn), lambda j: (0, j)),
        out_shape=jax.ShapeDtypeStruct((g, bk), jnp.float32),
        compiler_params=pltpu.CompilerParams(
            dimension_semantics=("arbitrary",),
        ),
    )(adj_norm, x)

    s = adj_spatial_norm
    sbig = jnp.concatenate(
        [s[0::2, 0::2], s[0::2, 1::2], s[1::2, 0::2], s[1::2, 1::2]], axis=0)
    z = jnp.zeros((_K, _K), jnp.float32)
    w1, w2, w3 = weight[:_K], weight[_K:2 * _K], weight[2 * _K:]
    bd1 = jnp.block([[w1, z], [z, w1]])
    bd2 = jnp.block([[w2, z], [z, w2]])
    bd3 = jnp.block([[w3, z], [z, w3]])
    o = jnp.full((_K, _K), 1.0 / _K, jnp.float32)
    mred = jnp.block([[o, z], [z, o]])
    gam2 = jnp.concatenate([ln_gamma, ln_gamma]).reshape(1, _L)
    bet2 = jnp.concatenate([ln_beta, ln_beta]).reshape(1, _L)

    gblk = 8
    out = pl.pallas_call(
        _fused_body,
        grid=(g // gblk,),
        in_specs=[
            pl.BlockSpec((4 * _P, _P), lambda i: (0, 0)),
            pl.BlockSpec((_L, _L), lambda i: (0, 0)),
            pl.BlockSpec((_L, _L), lambda i: (0, 0)),
            pl.BlockSpec((_L, _L), lambda i: (0, 0)),
            pl.BlockSpec((_L, _L), lambda i: (0, 0)),
            pl.BlockSpec((1, _L), lambda i: (0, 0)),
            pl.BlockSpec((1, _L), lambda i: (0, 0)),
            pl.BlockSpec((gblk, bk), lambda i: (i, 0)),
            pl.BlockSpec((gblk, bk), lambda i: (i, 0)),
        ],
        out_specs=pl.BlockSpec((gblk, bk), lambda i: (i, 0)),
        out_shape=jax.ShapeDtypeStruct((g, bk), jnp.float32),
        compiler_params=pltpu.CompilerParams(
            dimension_semantics=("arbitrary",),
        ),
    )(sbig, bd1, bd2, bd3, mred, gam2, bet2, x, ng)
    return out


# speedup vs baseline: 3.2620x; 1.1331x over previous
import jax
import jax.numpy as jnp
from jax.experimental import pallas as pl
from jax.experimental.pallas import tpu as pltpu

_G = 1000
_B = 512
_K = 64
_P = _B // 2
_L = 2 * _K


def _matmul_body(a_ref, x_ref, o_ref):
    o_ref[...] = jnp.dot(a_ref[...], x_ref[...].astype(jnp.bfloat16),
                         preferred_element_type=jnp.float32
                         ).astype(jnp.bfloat16)


def _fused_body(sbig_ref, bd1_ref, bd2_ref, bd3_ref, mred_ref,
                gam_ref, bet_ref, x_ref, ng_ref, o_ref):
    gb = x_ref.shape[0]
    bk = x_ref.shape[1]
    rows = gb * _P

    x2 = x_ref[...].reshape(rows, _L)
    x2h = x2.astype(jnp.bfloat16)
    ng2h = ng_ref[...].reshape(rows, _L)

    lin = jnp.dot(x2h, bd1_ref[...], preferred_element_type=jnp.float32)
    lin = lin + jnp.dot(ng2h, bd2_ref[...], preferred_element_type=jnp.float32)
    p = jnp.dot(x2h, bd3_ref[...], preferred_element_type=jnp.float32)

    p3 = p.astype(jnp.bfloat16).reshape(gb, _P, _L)
    sbig = sbig_ref[...]
    ns_list = []
    for i in range(gb):
        q = jnp.dot(sbig, p3[i], preferred_element_type=jnp.float32)
        lo = q[0:_P, 0:_K] + q[_P:2 * _P, _K:_L]
        hi = q[2 * _P:3 * _P, 0:_K] + q[3 * _P:4 * _P, _K:_L]
        ns_list.append(jnp.concatenate([lo, hi], axis=1))
    ns = jnp.stack(ns_list).reshape(rows, _L)

    h = jax.nn.relu(lin + ns)
    r = x2 + h

    mred = mred_ref[...]
    mu = jnp.dot(r, mred, preferred_element_type=jnp.float32)
    var = jnp.dot(r * r, mred, preferred_element_type=jnp.float32) - mu * mu
    o = (r - mu) * jax.lax.rsqrt(var + 1e-5) * gam_ref[...] + bet_ref[...]
    o_ref[...] = o.reshape(gb, bk)


def kernel(x, adj_norm, adj_spatial_norm, weight, ln_gamma, ln_beta):
    g, bk = x.shape
    cn = 2048
    ng = pl.pallas_call(
        _matmul_body,
        grid=(bk // cn,),
        in_specs=[
            pl.BlockSpec((g, g), lambda j: (0, 0)),
            pl.BlockSpec((g, cn), lambda j: (0, j)),
        ],
        out_specs=pl.BlockSpec((g, cn), lambda j: (0, j)),
        out_shape=jax.ShapeDtypeStruct((g, bk), jnp.bfloat16),
        compiler_params=pltpu.CompilerParams(
            dimension_semantics=("arbitrary",),
        ),
    )(adj_norm.astype(jnp.bfloat16), x)

    s = adj_spatial_norm
    sbig = jnp.concatenate(
        [s[0::2, 0::2], s[0::2, 1::2], s[1::2, 0::2], s[1::2, 1::2]],
        axis=0).astype(jnp.bfloat16)
    z = jnp.zeros((_K, _K), jnp.float32)
    w1, w2, w3 = weight[:_K], weight[_K:2 * _K], weight[2 * _K:]
    bd1 = jnp.block([[w1, z], [z, w1]]).astype(jnp.bfloat16)
    bd2 = jnp.block([[w2, z], [z, w2]]).astype(jnp.bfloat16)
    bd3 = jnp.block([[w3, z], [z, w3]]).astype(jnp.bfloat16)
    o = jnp.full((_K, _K), 1.0 / _K, jnp.float32)
    mred = jnp.block([[o, z], [z, o]])
    gam2 = jnp.concatenate([ln_gamma, ln_gamma]).reshape(1, _L)
    bet2 = jnp.concatenate([ln_beta, ln_beta]).reshape(1, _L)

    gblk = 40
    out = pl.pallas_call(
        _fused_body,
        grid=(g // gblk,),
        in_specs=[
            pl.BlockSpec((4 * _P, _P), lambda i: (0, 0)),
            pl.BlockSpec((_L, _L), lambda i: (0, 0)),
            pl.BlockSpec((_L, _L), lambda i: (0, 0)),
            pl.BlockSpec((_L, _L), lambda i: (0, 0)),
            pl.BlockSpec((_L, _L), lambda i: (0, 0)),
            pl.BlockSpec((1, _L), lambda i: (0, 0)),
            pl.BlockSpec((1, _L), lambda i: (0, 0)),
            pl.BlockSpec((gblk, bk), lambda i: (i, 0)),
            pl.BlockSpec((gblk, bk), lambda i: (i, 0)),
        ],
        out_specs=pl.BlockSpec((gblk, bk), lambda i: (i, 0)),
        out_shape=jax.ShapeDtypeStruct((g, bk), jnp.float32),
        compiler_params=pltpu.CompilerParams(
            dimension_semantics=("arbitrary",),
        ),
    )(sbig, bd1, bd2, bd3, mred, gam2, bet2, x, ng)
    return out
